# baseline (speedup 1.0000x reference)
import functools

import jax
import jax.numpy as jnp
from jax import lax
from jax.experimental import pallas as pl
from jax.experimental.pallas import tpu as pltpu
from jax.experimental.pallas import tpu_sc as plsc

_B = 256
_CHUNK = 64
_CCH = 32


def _route_kernel(x_ref, gw_ref, bias_ref, pos_ref, w_ref, bexp_ref, nact_ref,
                  *, n_blocks):
    x = x_ref[...]
    gw = gw_ref[...]
    t, e = x.shape[0], gw.shape[0]
    logits = lax.dot_general(
        x.astype(jnp.bfloat16), gw.astype(jnp.bfloat16), (((1,), (1,)), ((), ())),
        preferred_element_type=jnp.float32)
    scores = jax.nn.sigmoid(logits)
    sfc = scores + bias_ref[...]
    eidx = lax.broadcasted_iota(jnp.int32, (t, e), 1)
    m1 = jnp.max(sfc, axis=1, keepdims=True)
    i1 = jnp.min(jnp.where(sfc == m1, eidx, e), axis=1, keepdims=True)
    oh1 = eidx == i1
    w1 = jnp.sum(jnp.where(oh1, scores, 0.0), axis=1, keepdims=True)
    sfc2 = jnp.where(oh1, -1e30, sfc)
    m2 = jnp.max(sfc2, axis=1, keepdims=True)
    i2 = jnp.min(jnp.where(sfc2 == m2, eidx, e), axis=1, keepdims=True)
    oh2 = eidx == i2
    w2 = jnp.sum(jnp.where(oh2, scores, 0.0), axis=1, keepdims=True)
    denom = w1 + w2 + 1e-20

    oh = (oh1 | oh2).astype(jnp.float32)
    C = 256
    lt = (lax.broadcasted_iota(jnp.int32, (C, C), 0)
          > lax.broadcasted_iota(jnp.int32, (C, C), 1)).astype(jnp.float32)
    run = jnp.zeros((1, e), jnp.float32)
    cums = []
    for c in range(t // C):
        ohc = oh[c * C:(c + 1) * C]
        exc = lax.dot_general(lt, ohc, (((1,), (0,)), ((), ())),
                              preferred_element_type=jnp.float32) + run
        cums.append(exc)
        run = run + jnp.sum(ohc, axis=0, keepdims=True)
    cum = jnp.concatenate(cums, axis=0)
    counts = run

    bf = jnp.float32(_B)
    nblk_row = jnp.floor((counts + (bf - 1.0)) / bf)
    m_le = (lax.broadcasted_iota(jnp.int32, (e, e), 0)
            <= lax.broadcasted_iota(jnp.int32, (e, e), 1)).astype(jnp.float32)
    cumincl = lax.dot_general(nblk_row, m_le, (((1,), (0,)), ((), ())),
                              preferred_element_type=jnp.float32)
    gs_row = (cumincl - nblk_row) * bf

    base = gs_row + cum
    pos0 = jnp.sum(jnp.where(oh1, base, 0.0), axis=1, keepdims=True)
    pos1 = jnp.sum(jnp.where(oh2, base, 0.0), axis=1, keepdims=True)
    pos_ref[...] = jnp.concatenate([pos0, pos1], axis=1).astype(jnp.int32)
    w_ref[...] = jnp.concatenate([w1 / denom, w2 / denom], axis=1)

    bid = lax.broadcasted_iota(jnp.int32, (n_blocks, 1), 0).astype(jnp.float32)
    raw = jnp.sum((bid >= cumincl).astype(jnp.float32), axis=1, keepdims=True)
    bexp_ref[...] = jnp.minimum(raw, jnp.float32(e - 1)).astype(jnp.int32)
    nact_ref[...] = cumincl[:, e - 1:e].astype(jnp.int32)


def _grouped_kernel(bexp_ref, nact_ref, xs_ref, w1_ref, w2_ref, sw_ref, ys_ref):
    b = pl.program_id(0)

    @pl.when(b < nact_ref[0])
    def _():
        a = lax.dot_general(xs_ref[...].astype(jnp.bfloat16),
                            w1_ref[0].astype(jnp.bfloat16),
                            (((1,), (1,)), ((), ())),
                            preferred_element_type=jnp.float32)
        h = jnp.square(jnp.maximum(a, 0.0)).astype(jnp.bfloat16)
        y = lax.dot_general(h, w2_ref[0].astype(jnp.bfloat16),
                            (((1,), (1,)), ((), ())),
                            preferred_element_type=jnp.float32)
        ys_ref[...] = y * sw_ref[0]


def _shared_kernel(x_ref, w1_ref, w2_ref, out_ref):
    a = lax.dot_general(x_ref[...].astype(jnp.bfloat16),
                        w1_ref[...].astype(jnp.bfloat16),
                        (((1,), (1,)), ((), ())),
                        preferred_element_type=jnp.float32)
    h = jnp.square(jnp.maximum(a, 0.0)).astype(jnp.bfloat16)
    out_ref[...] = lax.dot_general(h, w2_ref[...].astype(jnp.bfloat16),
                                   (((1,), (1,)), ((), ())),
                                   preferred_element_type=jnp.float32)


def _combine_kernel(sh_ref, y0_ref, y1_ref, out_ref):
    out_ref[...] = sh_ref[...] + y0_ref[...] + y1_ref[...]


def kernel(hidden_states, gate_weight, e_score_correction_bias, shared_w1,
           shared_w2, expert_w1, expert_w2):
    T, H = hidden_states.shape
    E, I_, _ = expert_w1.shape
    SI = shared_w1.shape[0]
    NB = (T * 2) // _B + E
    S_pad = NB * _B
    SL = H // 128

    x = hidden_states
    pos, wts, bexp2, nact2 = pl.pallas_call(
        functools.partial(_route_kernel, n_blocks=NB),
        out_shape=(
            jax.ShapeDtypeStruct((T, 2), jnp.int32),
            jax.ShapeDtypeStruct((T, 2), jnp.float32),
            jax.ShapeDtypeStruct((NB, 1), jnp.int32),
            jax.ShapeDtypeStruct((1, 1), jnp.int32),
        ),
    )(x, gate_weight, e_score_correction_bias.reshape(1, E))

    pos_flat = pos.T.reshape(-1)
    w_flat = wts.T.reshape(-1)

    info = plsc.get_sparse_core_info()
    NW = info.num_cores * info.num_subcores
    mesh = plsc.VectorSubcoreMesh(core_axis_name="c", subcore_axis_name="s")

    SUB = 16
    NSUB = _CHUNK // SUB

    @functools.partial(
        pl.kernel, mesh=mesh,
        out_type=(
            jax.ShapeDtypeStruct((S_pad, H), jnp.float32),
            jax.ShapeDtypeStruct((S_pad,), jnp.float32),
        ),
        scratch_types=[
            pltpu.VMEM((SUB, H), jnp.float32),
            pltpu.VMEM((SUB, H), jnp.float32),
            pltpu.VMEM((SUB,), jnp.int32),
            pltpu.VMEM((SUB,), jnp.int32),
            pltpu.VMEM((SUB,), jnp.int32),
            pltpu.VMEM((SUB,), jnp.int32),
            pltpu.VMEM((SUB,), jnp.float32),
            pltpu.VMEM((SUB,), jnp.float32),
            pltpu.VMEM((SUB,), jnp.float32),
            pltpu.VMEM((SUB,), jnp.float32),
            pltpu.SemaphoreType.DMA,
            pltpu.SemaphoreType.DMA,
        ],
    )
    def _dispatch(x_hbm, pos_hbm, w_hbm, xs_hbm, sw_hbm,
                  xv0, xv1, i0a, i0b, i1a, i1b, w0a, w0b, w1a, w1b, s0, s1):
        wid = lax.axis_index("s") * info.num_cores + lax.axis_index("c")
        xv, i0, i1 = (xv0, xv1), (i0a, i0b), (i1a, i1b)
        w0, w1 = (w0a, w0b), (w1a, w1b)
        sems = (s0, s1)
        pend = [None, None]
        for j in range(NSUB):
            sl = j % 2
            if pend[sl]:
                for hnd in pend[sl]:
                    hnd.wait()
            base = wid * _CHUNK + j * SUB
            pltpu.sync_copy(x_hbm.at[pl.ds(base, SUB)], xv[sl])
            pltpu.sync_copy(pos_hbm.at[pl.ds(base, SUB)], i0[sl])
            pltpu.sync_copy(pos_hbm.at[pl.ds(T + base, SUB)], i1[sl])
            pltpu.sync_copy(w_hbm.at[pl.ds(base, SUB)], w0[sl])
            pltpu.sync_copy(w_hbm.at[pl.ds(T + base, SUB)], w1[sl])
            pend[sl] = [
                pltpu.async_copy(xv[sl], xs_hbm.at[i0[sl]], sems[sl]),
                pltpu.async_copy(xv[sl], xs_hbm.at[i1[sl]], sems[sl]),
                pltpu.async_copy(w0[sl], sw_hbm.at[i0[sl]], sems[sl]),
                pltpu.async_copy(w1[sl], sw_hbm.at[i1[sl]], sems[sl]),
            ]
        for p in pend:
            if p:
                for hnd in p:
                    hnd.wait()

    xs2, slot_w = _dispatch(x, pos_flat, w_flat)

    TB = 256

    def _shared_half(x_half):
        return pl.pallas_call(
            _shared_kernel,
            grid=(T // 2 // TB,),
            in_specs=[
                pl.BlockSpec((TB, H), lambda i: (i, 0)),
                pl.BlockSpec((SI, H), lambda i: (0, 0)),
                pl.BlockSpec((H, SI), lambda i: (0, 0)),
            ],
            out_specs=pl.BlockSpec((TB, H), lambda i: (i, 0)),
            out_shape=jax.ShapeDtypeStruct((T // 2, H), jnp.float32),
        )(x_half, shared_w1, shared_w2)

    sh_a = _shared_half(lax.slice(x, (0, 0), (T // 2, H)))

    ys = pl.pallas_call(
        _grouped_kernel,
        grid_spec=pltpu.PrefetchScalarGridSpec(
            num_scalar_prefetch=2,
            grid=(NB,),
            in_specs=[
                pl.BlockSpec((_B, H), lambda b, be, na: (b, 0)),
                pl.BlockSpec((1, I_, H), lambda b, be, na: (be[b], 0, 0)),
                pl.BlockSpec((1, H, I_), lambda b, be, na: (be[b], 0, 0)),
                pl.BlockSpec((1, _B, 1), lambda b, be, na: (b, 0, 0)),
            ],
            out_specs=pl.BlockSpec((_B, H), lambda b, be, na: (b, 0)),
        ),
        out_shape=jax.ShapeDtypeStruct((S_pad, H), jnp.float32),
        compiler_params=pltpu.CompilerParams(
            dimension_semantics=("arbitrary",),
        ),
    )(bexp2.reshape(NB), nact2.reshape(1), xs2,
      expert_w1, expert_w2, slot_w.reshape(NB, _B, 1))

    @functools.partial(
        pl.kernel, mesh=mesh,
        out_type=jax.ShapeDtypeStruct((2 * T, H), jnp.float32),
        scratch_types=[
            pltpu.VMEM((SUB, H), jnp.float32),
            pltpu.VMEM((SUB, H), jnp.float32),
            pltpu.VMEM((SUB,), jnp.int32),
            pltpu.VMEM((SUB,), jnp.int32),
            pltpu.SemaphoreType.DMA,
            pltpu.SemaphoreType.DMA,
            pltpu.SemaphoreType.DMA,
            pltpu.SemaphoreType.DMA,
        ],
    )
    def _ygather(pos_hbm, ys_hbm, yt_hbm, b0, b1, iv0, iv1, g0, g1, st0, st1):
        wid = lax.axis_index("s") * info.num_cores + lax.axis_index("c")
        bufs, ivs = (b0, b1), (iv0, iv1)
        gsem, ssem = (g0, g1), (st0, st1)
        pend = [None, None]
        for k in range(2):
            for j in range(NSUB):
                jj = k * NSUB + j
                sl = jj % 2
                if pend[sl]:
                    pend[sl].wait()
                b2 = k * T + wid * _CHUNK + j * SUB
                pltpu.sync_copy(pos_hbm.at[pl.ds(b2, SUB)], ivs[sl])
                pltpu.async_copy(ys_hbm.at[ivs[sl]], bufs[sl], gsem[sl]).wait()
                pend[sl] = pltpu.async_copy(bufs[sl], yt_hbm.at[pl.ds(b2, SUB)],
                                            ssem[sl])
        for p in pend:
            if p:
                p.wait()

    yt = _ygather(pos_flat, ys)

    sh_b = _shared_half(lax.slice(x, (T // 2, 0), (T, H)))
    sh = jnp.concatenate([sh_a, sh_b], axis=0)

    CB = 512
    return pl.pallas_call(
        _combine_kernel,
        grid=(T // CB,),
        in_specs=[
            pl.BlockSpec((CB, H), lambda i: (i, 0)),
            pl.BlockSpec((CB, H), lambda i: (i, 0)),
            pl.BlockSpec((CB, H), lambda i: (i + T // CB, 0)),
        ],
        out_specs=pl.BlockSpec((CB, H), lambda i: (i, 0)),
        out_shape=jax.ShapeDtypeStruct((T, H), jnp.float32),
    )(sh, yt, yt)


# speedup vs baseline: 1.1715x; 1.1715x over previous
import functools

import jax
import jax.numpy as jnp
from jax import lax
from jax.experimental import pallas as pl
from jax.experimental.pallas import tpu as pltpu
from jax.experimental.pallas import tpu_sc as plsc

_B = 256
_CHUNK = 64
_CCH = 32


def _route_kernel(x_ref, gw_ref, bias_ref, pos_ref, w_ref, bexp_ref, nact_ref,
                  *, n_blocks):
    x = x_ref[...]
    gw = gw_ref[...]
    t, e = x.shape[0], gw.shape[0]
    logits = lax.dot_general(
        x.astype(jnp.bfloat16), gw.astype(jnp.bfloat16), (((1,), (1,)), ((), ())),
        preferred_element_type=jnp.float32)
    scores = jax.nn.sigmoid(logits)
    sfc = scores + bias_ref[...]
    eidx = lax.broadcasted_iota(jnp.int32, (t, e), 1)
    m1 = jnp.max(sfc, axis=1, keepdims=True)
    i1 = jnp.min(jnp.where(sfc == m1, eidx, e), axis=1, keepdims=True)
    oh1 = eidx == i1
    w1 = jnp.sum(jnp.where(oh1, scores, 0.0), axis=1, keepdims=True)
    sfc2 = jnp.where(oh1, -1e30, sfc)
    m2 = jnp.max(sfc2, axis=1, keepdims=True)
    i2 = jnp.min(jnp.where(sfc2 == m2, eidx, e), axis=1, keepdims=True)
    oh2 = eidx == i2
    w2 = jnp.sum(jnp.where(oh2, scores, 0.0), axis=1, keepdims=True)
    denom = w1 + w2 + 1e-20

    oh = (oh1 | oh2).astype(jnp.float32)
    C = 256
    lt = (lax.broadcasted_iota(jnp.int32, (C, C), 0)
          > lax.broadcasted_iota(jnp.int32, (C, C), 1)).astype(jnp.float32)
    run = jnp.zeros((1, e), jnp.float32)
    cums = []
    for c in range(t // C):
        ohc = oh[c * C:(c + 1) * C]
        exc = lax.dot_general(lt, ohc, (((1,), (0,)), ((), ())),
                              preferred_element_type=jnp.float32) + run
        cums.append(exc)
        run = run + jnp.sum(ohc, axis=0, keepdims=True)
    cum = jnp.concatenate(cums, axis=0)
    counts = run

    bf = jnp.float32(_B)
    nblk_row = jnp.floor((counts + (bf - 1.0)) / bf)
    m_le = (lax.broadcasted_iota(jnp.int32, (e, e), 0)
            <= lax.broadcasted_iota(jnp.int32, (e, e), 1)).astype(jnp.float32)
    cumincl = lax.dot_general(nblk_row, m_le, (((1,), (0,)), ((), ())),
                              preferred_element_type=jnp.float32)
    gs_row = (cumincl - nblk_row) * bf

    base = gs_row + cum
    pos0 = jnp.sum(jnp.where(oh1, base, 0.0), axis=1, keepdims=True)
    pos1 = jnp.sum(jnp.where(oh2, base, 0.0), axis=1, keepdims=True)
    pos_ref[...] = jnp.concatenate([pos0, pos1], axis=1).astype(jnp.int32)
    w_ref[...] = jnp.concatenate([w1 / denom, w2 / denom], axis=1)

    bid = lax.broadcasted_iota(jnp.int32, (n_blocks, 1), 0).astype(jnp.float32)
    raw = jnp.sum((bid >= cumincl).astype(jnp.float32), axis=1, keepdims=True)
    bexp_ref[...] = jnp.minimum(raw, jnp.float32(e - 1)).astype(jnp.int32)
    nact_ref[...] = cumincl[:, e - 1:e].astype(jnp.int32)


def _grouped_kernel(bexp_ref, nact_ref, xs_ref, w1_ref, w2_ref, sw_ref, ys_ref):
    b = pl.program_id(0)

    @pl.when(b < nact_ref[0])
    def _():
        nb, hp = xs_ref.shape
        xb = pltpu.bitcast(xs_ref[...], jnp.bfloat16).reshape(nb, 2 * hp)
        a = lax.dot_general(xb, w1_ref[0].astype(jnp.bfloat16),
                            (((1,), (1,)), ((), ())),
                            preferred_element_type=jnp.float32)
        h = jnp.square(jnp.maximum(a, 0.0)).astype(jnp.bfloat16)
        y = lax.dot_general(h, w2_ref[0].astype(jnp.bfloat16),
                            (((1,), (1,)), ((), ())),
                            preferred_element_type=jnp.float32)
        yw = (y * sw_ref[0]).astype(jnp.bfloat16)
        ys_ref[...] = pltpu.bitcast(yw.reshape(2 * nb, hp), jnp.int32)


def _shared_combine_kernel(x_ref, w1_ref, w2_ref, y0_ref, y1_ref, out_ref):
    a = lax.dot_general(x_ref[...].astype(jnp.bfloat16),
                        w1_ref[...].astype(jnp.bfloat16),
                        (((1,), (1,)), ((), ())),
                        preferred_element_type=jnp.float32)
    h = jnp.square(jnp.maximum(a, 0.0)).astype(jnp.bfloat16)
    s = lax.dot_general(h, w2_ref[...].astype(jnp.bfloat16),
                        (((1,), (1,)), ((), ())),
                        preferred_element_type=jnp.float32)
    tb, hp = y0_ref.shape
    y0 = pltpu.bitcast(y0_ref[...], jnp.bfloat16).reshape(tb, 2 * hp)
    y1 = pltpu.bitcast(y1_ref[...], jnp.bfloat16).reshape(tb, 2 * hp)
    out_ref[...] = s + y0.astype(jnp.float32) + y1.astype(jnp.float32)


def kernel(hidden_states, gate_weight, e_score_correction_bias, shared_w1,
           shared_w2, expert_w1, expert_w2):
    T, H = hidden_states.shape
    E, I_, _ = expert_w1.shape
    SI = shared_w1.shape[0]
    NB = (T * 2) // _B + E
    S_pad = NB * _B
    SL = H // 128

    x = hidden_states
    pos, wts, bexp2, nact2 = pl.pallas_call(
        functools.partial(_route_kernel, n_blocks=NB),
        out_shape=(
            jax.ShapeDtypeStruct((T, 2), jnp.int32),
            jax.ShapeDtypeStruct((T, 2), jnp.float32),
            jax.ShapeDtypeStruct((NB, 1), jnp.int32),
            jax.ShapeDtypeStruct((1, 1), jnp.int32),
        ),
    )(x, gate_weight, e_score_correction_bias.reshape(1, E))

    pos_flat = pos.T.reshape(-1)
    w_flat = wts.T.reshape(-1)
    HP = H // 2
    xb16 = x.astype(jnp.bfloat16)
    xp = lax.bitcast_convert_type(
        jnp.stack([xb16[:, :HP], xb16[:, HP:]], axis=-1), jnp.int32)

    info = plsc.get_sparse_core_info()
    NW = info.num_cores * info.num_subcores
    mesh = plsc.VectorSubcoreMesh(core_axis_name="c", subcore_axis_name="s")

    SUB = 16
    NSUB = _CHUNK // SUB

    @functools.partial(
        pl.kernel, mesh=mesh,
        out_type=(
            jax.ShapeDtypeStruct((S_pad, HP), jnp.int32),
            jax.ShapeDtypeStruct((S_pad,), jnp.float32),
        ),
        scratch_types=[
            pltpu.VMEM((SUB, HP), jnp.int32),
            pltpu.VMEM((SUB, HP), jnp.int32),
            pltpu.VMEM((SUB,), jnp.int32),
            pltpu.VMEM((SUB,), jnp.int32),
            pltpu.VMEM((SUB,), jnp.int32),
            pltpu.VMEM((SUB,), jnp.int32),
            pltpu.VMEM((SUB,), jnp.float32),
            pltpu.VMEM((SUB,), jnp.float32),
            pltpu.VMEM((SUB,), jnp.float32),
            pltpu.VMEM((SUB,), jnp.float32),
            pltpu.SemaphoreType.DMA,
            pltpu.SemaphoreType.DMA,
        ],
    )
    def _dispatch(x_hbm, pos_hbm, w_hbm, xs_hbm, sw_hbm,
                  xv0, xv1, i0a, i0b, i1a, i1b, w0a, w0b, w1a, w1b, s0, s1):
        wid = lax.axis_index("s") * info.num_cores + lax.axis_index("c")
        xv, i0, i1 = (xv0, xv1), (i0a, i0b), (i1a, i1b)
        w0, w1 = (w0a, w0b), (w1a, w1b)
        sems = (s0, s1)
        pend = [None, None]
        for j in range(NSUB):
            sl = j % 2
            if pend[sl]:
                for hnd in pend[sl]:
                    hnd.wait()
            base = wid * _CHUNK + j * SUB
            pltpu.sync_copy(x_hbm.at[pl.ds(base, SUB)], xv[sl])
            pltpu.sync_copy(pos_hbm.at[pl.ds(base, SUB)], i0[sl])
            pltpu.sync_copy(pos_hbm.at[pl.ds(T + base, SUB)], i1[sl])
            pltpu.sync_copy(w_hbm.at[pl.ds(base, SUB)], w0[sl])
            pltpu.sync_copy(w_hbm.at[pl.ds(T + base, SUB)], w1[sl])
            pend[sl] = [
                pltpu.async_copy(xv[sl], xs_hbm.at[i0[sl]], sems[sl]),
                pltpu.async_copy(xv[sl], xs_hbm.at[i1[sl]], sems[sl]),
                pltpu.async_copy(w0[sl], sw_hbm.at[i0[sl]], sems[sl]),
                pltpu.async_copy(w1[sl], sw_hbm.at[i1[sl]], sems[sl]),
            ]
        for p in pend:
            if p:
                for hnd in p:
                    hnd.wait()

    xs2, slot_w = _dispatch(xp, pos_flat, w_flat)

    ys = pl.pallas_call(
        _grouped_kernel,
        grid_spec=pltpu.PrefetchScalarGridSpec(
            num_scalar_prefetch=2,
            grid=(NB,),
            in_specs=[
                pl.BlockSpec((_B, HP), lambda b, be, na: (b, 0)),
                pl.BlockSpec((1, I_, H), lambda b, be, na: (be[b], 0, 0)),
                pl.BlockSpec((1, H, I_), lambda b, be, na: (be[b], 0, 0)),
                pl.BlockSpec((1, _B, 1), lambda b, be, na: (b, 0, 0)),
            ],
            out_specs=pl.BlockSpec((_B, HP), lambda b, be, na: (b, 0)),
        ),
        out_shape=jax.ShapeDtypeStruct((S_pad, HP), jnp.int32),
        compiler_params=pltpu.CompilerParams(
            dimension_semantics=("arbitrary",),
        ),
    )(bexp2.reshape(NB), nact2.reshape(1), xs2,
      expert_w1, expert_w2, slot_w.reshape(NB, _B, 1))

    @functools.partial(
        pl.kernel, mesh=mesh,
        out_type=jax.ShapeDtypeStruct((2 * T, HP), jnp.int32),
        scratch_types=[
            pltpu.VMEM((SUB, HP), jnp.int32),
            pltpu.VMEM((SUB, HP), jnp.int32),
            pltpu.VMEM((SUB,), jnp.int32),
            pltpu.VMEM((SUB,), jnp.int32),
            pltpu.SemaphoreType.DMA,
            pltpu.SemaphoreType.DMA,
            pltpu.SemaphoreType.DMA,
            pltpu.SemaphoreType.DMA,
        ],
    )
    def _ygather(pos_hbm, ys_hbm, yt_hbm, b0, b1, iv0, iv1, g0, g1, st0, st1):
        wid = lax.axis_index("s") * info.num_cores + lax.axis_index("c")
        bufs, ivs = (b0, b1), (iv0, iv1)
        gsem, ssem = (g0, g1), (st0, st1)
        pend = [None, None]
        for k in range(2):
            for j in range(NSUB):
                jj = k * NSUB + j
                sl = jj % 2
                if pend[sl]:
                    pend[sl].wait()
                b2 = k * T + wid * _CHUNK + j * SUB
                pltpu.sync_copy(pos_hbm.at[pl.ds(b2, SUB)], ivs[sl])
                pltpu.async_copy(ys_hbm.at[ivs[sl]], bufs[sl], gsem[sl]).wait()
                pend[sl] = pltpu.async_copy(bufs[sl], yt_hbm.at[pl.ds(b2, SUB)],
                                            ssem[sl])
        for p in pend:
            if p:
                p.wait()

    yt = _ygather(pos_flat, ys)

    TB = 256
    return pl.pallas_call(
        _shared_combine_kernel,
        grid=(T // TB,),
        in_specs=[
            pl.BlockSpec((TB, H), lambda i: (i, 0)),
            pl.BlockSpec((SI, H), lambda i: (0, 0)),
            pl.BlockSpec((H, SI), lambda i: (0, 0)),
            pl.BlockSpec((TB, HP), lambda i: (i, 0)),
            pl.BlockSpec((TB, HP), lambda i: (i + T // TB, 0)),
        ],
        out_specs=pl.BlockSpec((TB, H), lambda i: (i, 0)),
        out_shape=jax.ShapeDtypeStruct((T, H), jnp.float32),
    )(x, shared_w1, shared_w2, yt, yt)


# speedup vs baseline: 1.2743x; 1.0878x over previous
import functools

import jax
import jax.numpy as jnp
from jax import lax
from jax.experimental import pallas as pl
from jax.experimental.pallas import tpu as pltpu
from jax.experimental.pallas import tpu_sc as plsc

_B = 256
_CHUNK = 64
_CCH = 32


def _route_kernel(x_ref, gw_ref, bias_ref, pos_ref, w_ref, bexp_ref, nact_ref,
                  xp_ref, *, n_blocks):
    x = x_ref[...]
    gw = gw_ref[...]
    t, e = x.shape[0], gw.shape[0]
    hp = x.shape[1] // 2
    xp_ref[...] = pltpu.bitcast(
        x.astype(jnp.bfloat16).reshape(2 * t, hp), jnp.int32)
    logits = lax.dot_general(
        x.astype(jnp.bfloat16), gw.astype(jnp.bfloat16), (((1,), (1,)), ((), ())),
        preferred_element_type=jnp.float32)
    scores = jax.nn.sigmoid(logits)
    sfc = scores + bias_ref[...]
    eidx = lax.broadcasted_iota(jnp.int32, (t, e), 1)
    m1 = jnp.max(sfc, axis=1, keepdims=True)
    i1 = jnp.min(jnp.where(sfc == m1, eidx, e), axis=1, keepdims=True)
    oh1 = eidx == i1
    w1 = jnp.sum(jnp.where(oh1, scores, 0.0), axis=1, keepdims=True)
    sfc2 = jnp.where(oh1, -1e30, sfc)
    m2 = jnp.max(sfc2, axis=1, keepdims=True)
    i2 = jnp.min(jnp.where(sfc2 == m2, eidx, e), axis=1, keepdims=True)
    oh2 = eidx == i2
    w2 = jnp.sum(jnp.where(oh2, scores, 0.0), axis=1, keepdims=True)
    denom = w1 + w2 + 1e-20

    oh = (oh1 | oh2).astype(jnp.float32)
    C = 256
    lt = (lax.broadcasted_iota(jnp.int32, (C, C), 0)
          > lax.broadcasted_iota(jnp.int32, (C, C), 1)).astype(jnp.float32)
    run = jnp.zeros((1, e), jnp.float32)
    cums = []
    for c in range(t // C):
        ohc = oh[c * C:(c + 1) * C]
        exc = lax.dot_general(lt, ohc, (((1,), (0,)), ((), ())),
                              preferred_element_type=jnp.float32) + run
        cums.append(exc)
        run = run + jnp.sum(ohc, axis=0, keepdims=True)
    cum = jnp.concatenate(cums, axis=0)
    counts = run

    bf = jnp.float32(_B)
    nblk_row = jnp.floor((counts + (bf - 1.0)) / bf)
    m_le = (lax.broadcasted_iota(jnp.int32, (e, e), 0)
            <= lax.broadcasted_iota(jnp.int32, (e, e), 1)).astype(jnp.float32)
    cumincl = lax.dot_general(nblk_row, m_le, (((1,), (0,)), ((), ())),
                              preferred_element_type=jnp.float32)
    gs_row = (cumincl - nblk_row) * bf

    base = gs_row + cum
    pos0 = jnp.sum(jnp.where(oh1, base, 0.0), axis=1, keepdims=True)
    pos1 = jnp.sum(jnp.where(oh2, base, 0.0), axis=1, keepdims=True)
    pos_ref[...] = jnp.concatenate([pos0, pos1], axis=1).astype(jnp.int32)
    w_ref[...] = jnp.concatenate([w1 / denom, w2 / denom], axis=1)

    bid = lax.broadcasted_iota(jnp.int32, (n_blocks, 1), 0).astype(jnp.float32)
    raw = jnp.sum((bid >= cumincl).astype(jnp.float32), axis=1, keepdims=True)
    bexp_ref[...] = jnp.minimum(raw, jnp.float32(e - 1)).astype(jnp.int32)
    nact_ref[...] = cumincl[:, e - 1:e].astype(jnp.int32)


def _grouped_kernel(bexp_ref, nact_ref, xs_ref, w1_ref, w2_ref, sw_ref, ys_ref):
    b = pl.program_id(0)

    @pl.when(b < nact_ref[0])
    def _():
        nb, hp = xs_ref.shape
        xb = pltpu.bitcast(xs_ref[...], jnp.bfloat16).reshape(nb, 2 * hp)
        a = lax.dot_general(xb, w1_ref[0].astype(jnp.bfloat16),
                            (((1,), (1,)), ((), ())),
                            preferred_element_type=jnp.float32)
        h = jnp.square(jnp.maximum(a, 0.0)).astype(jnp.bfloat16)
        y = lax.dot_general(h, w2_ref[0].astype(jnp.bfloat16),
                            (((1,), (1,)), ((), ())),
                            preferred_element_type=jnp.float32)
        yw = (y * sw_ref[0]).astype(jnp.bfloat16)
        ys_ref[...] = pltpu.bitcast(yw.reshape(2 * nb, hp), jnp.int32)


def _shared_combine_kernel(x_ref, w1_ref, w2_ref, y0_ref, y1_ref, out_ref):
    a = lax.dot_general(x_ref[...].astype(jnp.bfloat16),
                        w1_ref[...].astype(jnp.bfloat16),
                        (((1,), (1,)), ((), ())),
                        preferred_element_type=jnp.float32)
    h = jnp.square(jnp.maximum(a, 0.0)).astype(jnp.bfloat16)
    s = lax.dot_general(h, w2_ref[...].astype(jnp.bfloat16),
                        (((1,), (1,)), ((), ())),
                        preferred_element_type=jnp.float32)
    tb, hp = y0_ref.shape
    y0 = pltpu.bitcast(y0_ref[...], jnp.bfloat16).reshape(tb, 2 * hp)
    y1 = pltpu.bitcast(y1_ref[...], jnp.bfloat16).reshape(tb, 2 * hp)
    out_ref[...] = s + y0.astype(jnp.float32) + y1.astype(jnp.float32)


def kernel(hidden_states, gate_weight, e_score_correction_bias, shared_w1,
           shared_w2, expert_w1, expert_w2):
    T, H = hidden_states.shape
    E, I_, _ = expert_w1.shape
    SI = shared_w1.shape[0]
    NB = (T * 2) // _B + E
    S_pad = NB * _B
    SL = H // 128

    x = hidden_states
    pos, wts, bexp2, nact2, xp = pl.pallas_call(
        functools.partial(_route_kernel, n_blocks=NB),
        out_shape=(
            jax.ShapeDtypeStruct((T, 2), jnp.int32),
            jax.ShapeDtypeStruct((T, 2), jnp.float32),
            jax.ShapeDtypeStruct((NB, 1), jnp.int32),
            jax.ShapeDtypeStruct((1, 1), jnp.int32),
            jax.ShapeDtypeStruct((T, H // 2), jnp.int32),
        ),
    )(x, gate_weight, e_score_correction_bias.reshape(1, E))

    pos_flat = pos.T.reshape(-1)
    w_flat = wts.T.reshape(-1)
    HP = H // 2

    info = plsc.get_sparse_core_info()
    NW = info.num_cores * info.num_subcores
    mesh = plsc.VectorSubcoreMesh(core_axis_name="c", subcore_axis_name="s")

    SUB = 16
    NSUB = _CHUNK // SUB

    @functools.partial(
        pl.kernel, mesh=mesh,
        out_type=(
            jax.ShapeDtypeStruct((S_pad, HP), jnp.int32),
            jax.ShapeDtypeStruct((S_pad,), jnp.float32),
        ),
        scratch_types=[
            pltpu.VMEM((SUB, HP), jnp.int32),
            pltpu.VMEM((SUB, HP), jnp.int32),
            pltpu.VMEM((SUB,), jnp.int32),
            pltpu.VMEM((SUB,), jnp.int32),
            pltpu.VMEM((SUB,), jnp.int32),
            pltpu.VMEM((SUB,), jnp.int32),
            pltpu.VMEM((SUB,), jnp.float32),
            pltpu.VMEM((SUB,), jnp.float32),
            pltpu.VMEM((SUB,), jnp.float32),
            pltpu.VMEM((SUB,), jnp.float32),
            pltpu.SemaphoreType.DMA,
            pltpu.SemaphoreType.DMA,
        ],
    )
    def _dispatch(x_hbm, pos_hbm, w_hbm, xs_hbm, sw_hbm,
                  xv0, xv1, i0a, i0b, i1a, i1b, w0a, w0b, w1a, w1b, s0, s1):
        wid = lax.axis_index("s") * info.num_cores + lax.axis_index("c")
        xv, i0, i1 = (xv0, xv1), (i0a, i0b), (i1a, i1b)
        w0, w1 = (w0a, w0b), (w1a, w1b)
        sems = (s0, s1)
        pend = [None, None]
        for j in range(NSUB):
            sl = j % 2
            if pend[sl]:
                for hnd in pend[sl]:
                    hnd.wait()
            base = wid * _CHUNK + j * SUB
            pltpu.sync_copy(x_hbm.at[pl.ds(base, SUB)], xv[sl])
            pltpu.sync_copy(pos_hbm.at[pl.ds(base, SUB)], i0[sl])
            pltpu.sync_copy(pos_hbm.at[pl.ds(T + base, SUB)], i1[sl])
            pltpu.sync_copy(w_hbm.at[pl.ds(base, SUB)], w0[sl])
            pltpu.sync_copy(w_hbm.at[pl.ds(T + base, SUB)], w1[sl])
            pend[sl] = [
                pltpu.async_copy(xv[sl], xs_hbm.at[i0[sl]], sems[sl]),
                pltpu.async_copy(xv[sl], xs_hbm.at[i1[sl]], sems[sl]),
                pltpu.async_copy(w0[sl], sw_hbm.at[i0[sl]], sems[sl]),
                pltpu.async_copy(w1[sl], sw_hbm.at[i1[sl]], sems[sl]),
            ]
        for p in pend:
            if p:
                for hnd in p:
                    hnd.wait()

    xs2, slot_w = _dispatch(xp, pos_flat, w_flat)

    ys = pl.pallas_call(
        _grouped_kernel,
        grid_spec=pltpu.PrefetchScalarGridSpec(
            num_scalar_prefetch=2,
            grid=(NB,),
            in_specs=[
                pl.BlockSpec((_B, HP), lambda b, be, na: (b, 0)),
                pl.BlockSpec((1, I_, H), lambda b, be, na: (be[b], 0, 0)),
                pl.BlockSpec((1, H, I_), lambda b, be, na: (be[b], 0, 0)),
                pl.BlockSpec((1, _B, 1), lambda b, be, na: (b, 0, 0)),
            ],
            out_specs=pl.BlockSpec((_B, HP), lambda b, be, na: (b, 0)),
        ),
        out_shape=jax.ShapeDtypeStruct((S_pad, HP), jnp.int32),
        compiler_params=pltpu.CompilerParams(
            dimension_semantics=("arbitrary",),
        ),
    )(bexp2.reshape(NB), nact2.reshape(1), xs2,
      expert_w1, expert_w2, slot_w.reshape(NB, _B, 1))

    @functools.partial(
        pl.kernel, mesh=mesh,
        out_type=jax.ShapeDtypeStruct((2 * T, HP), jnp.int32),
        scratch_types=[
            pltpu.VMEM((SUB, HP), jnp.int32),
            pltpu.VMEM((SUB, HP), jnp.int32),
            pltpu.VMEM((SUB,), jnp.int32),
            pltpu.VMEM((SUB,), jnp.int32),
            pltpu.SemaphoreType.DMA,
            pltpu.SemaphoreType.DMA,
            pltpu.SemaphoreType.DMA,
            pltpu.SemaphoreType.DMA,
        ],
    )
    def _ygather(pos_hbm, ys_hbm, yt_hbm, b0, b1, iv0, iv1, g0, g1, st0, st1):
        wid = lax.axis_index("s") * info.num_cores + lax.axis_index("c")
        bufs, ivs = (b0, b1), (iv0, iv1)
        gsem, ssem = (g0, g1), (st0, st1)
        pend = [None, None]
        for k in range(2):
            for j in range(NSUB):
                jj = k * NSUB + j
                sl = jj % 2
                if pend[sl]:
                    pend[sl].wait()
                b2 = k * T + wid * _CHUNK + j * SUB
                pltpu.sync_copy(pos_hbm.at[pl.ds(b2, SUB)], ivs[sl])
                pltpu.async_copy(ys_hbm.at[ivs[sl]], bufs[sl], gsem[sl]).wait()
                pend[sl] = pltpu.async_copy(bufs[sl], yt_hbm.at[pl.ds(b2, SUB)],
                                            ssem[sl])
        for p in pend:
            if p:
                p.wait()

    yt = _ygather(pos_flat, ys)

    TB = 256
    return pl.pallas_call(
        _shared_combine_kernel,
        grid=(T // TB,),
        in_specs=[
            pl.BlockSpec((TB, H), lambda i: (i, 0)),
            pl.BlockSpec((SI, H), lambda i: (0, 0)),
            pl.BlockSpec((H, SI), lambda i: (0, 0)),
            pl.BlockSpec((TB, HP), lambda i: (i, 0)),
            pl.BlockSpec((TB, HP), lambda i: (i + T // TB, 0)),
        ],
        out_specs=pl.BlockSpec((TB, H), lambda i: (i, 0)),
        out_shape=jax.ShapeDtypeStruct((T, H), jnp.float32),
    )(x, shared_w1, shared_w2, yt, yt)
